# baseline (speedup 1.0000x reference)
import jax
import jax.numpy as jnp
from jax import lax
from jax.experimental import pallas as pl
from jax.experimental.pallas import tpu as pltpu
from jax.experimental.pallas import tpu_sc as plsc

_ROW_INDICES = (5, 8, 7, 16, 256, 123)
_NUM_ROWS = len(_ROW_INDICES)
_DIM = 64
_LANES = 16

_SPAN_START = min(_ROW_INDICES) // 128 * 128
_SPAN_COLS = (max(_ROW_INDICES) // 128 + 1) * 128 - _SPAN_START
_COORDS = tuple(
    ((i - _SPAN_START) // _LANES, (i - _SPAN_START) % _LANES)
    for i in _ROW_INDICES
)
_WINDOWS = tuple(sorted({win for win, _ in _COORDS}))


def _gather_body(tbl_hbm, out_hbm, chunk_v, rows_v, sem):
    sid = lax.axis_index("s")

    @pl.when(sid == 0)
    def _():
        pltpu.async_copy(
            tbl_hbm.at[:, pl.ds(_SPAN_START, _SPAN_COLS)], chunk_v, sem
        ).wait()

        pos = lax.iota(jnp.int32, _LANES)
        for q in range(_DIM // _LANES):
            acc = [jnp.zeros((_LANES,), jnp.float32) for _ in _ROW_INDICES]
            for t in range(_LANES):
                c = q * _LANES + t
                wvals = {
                    win: chunk_v[c, pl.ds(win * _LANES, _LANES)]
                    for win in _WINDOWS
                }
                for j, (win, lane) in enumerate(_COORDS):
                    acc[j] = jnp.where(pos == t, wvals[win][lane], acc[j])
            for j in range(_NUM_ROWS):
                rows_v[j, pl.ds(q * _LANES, _LANES)] = acc[j]
        pltpu.sync_copy(rows_v, out_hbm)


def kernel(input):
    tbl = input.T
    mesh = plsc.VectorSubcoreMesh(
        core_axis_name="c", subcore_axis_name="s", num_cores=1
    )
    gather = pl.kernel(
        _gather_body,
        mesh=mesh,
        out_type=jax.ShapeDtypeStruct((_NUM_ROWS, _DIM), jnp.float32),
        scratch_types=[
            pltpu.VMEM((_DIM, _SPAN_COLS), jnp.float32),
            pltpu.VMEM((_NUM_ROWS, _DIM), jnp.float32),
            pltpu.SemaphoreType.DMA,
        ],
    )
    return gather(tbl)


# speedup vs baseline: 1.0309x; 1.0309x over previous
import jax
import jax.numpy as jnp
from jax import lax
from jax.experimental import pallas as pl
from jax.experimental.pallas import tpu as pltpu
from jax.experimental.pallas import tpu_sc as plsc

_ROW_INDICES = (5, 8, 7, 16, 256, 123)
_NUM_ROWS = len(_ROW_INDICES)
_DIM = 64
_LANES = 16

_CHUNK_STARTS = tuple(sorted({i - i % 128 for i in _ROW_INDICES}))
_NUM_CHUNKS = len(_CHUNK_STARTS)
_COORDS = tuple(
    (_CHUNK_STARTS.index(i - i % 128), (i % 128) // _LANES, i % _LANES)
    for i in _ROW_INDICES
)
_WINDOWS = tuple(sorted({(blk, win) for blk, win, _ in _COORDS}))


def _gather_body(tbl_hbm, out_hbm, chunks_v, rows_v, sem):
    sid = lax.axis_index("s")

    @pl.when(sid == 0)
    def _():
        copies = [
            pltpu.async_copy(
                tbl_hbm.at[:, pl.ds(start, 128)], chunks_v.at[k], sem
            )
            for k, start in enumerate(_CHUNK_STARTS)
        ]
        for c in copies:
            c.wait()

        pos = lax.iota(jnp.int32, _LANES)
        for q in range(_DIM // _LANES):

            def t_body(t, acc):
                c = q * _LANES + t
                wvals = {
                    (blk, win): chunks_v[blk, c, pl.ds(win * _LANES, _LANES)]
                    for blk, win in _WINDOWS
                }
                return tuple(
                    jnp.where(pos == t, wvals[(blk, win)][lane], a)
                    for a, (blk, win, lane) in zip(acc, _COORDS)
                )

            acc = lax.fori_loop(
                0,
                _LANES,
                t_body,
                tuple(jnp.zeros((_LANES,), jnp.float32) for _ in _ROW_INDICES),
            )
            for j in range(_NUM_ROWS):
                rows_v[j, pl.ds(q * _LANES, _LANES)] = acc[j]
        pltpu.sync_copy(rows_v, out_hbm)


def kernel(input):
    tbl = input.T
    mesh = plsc.VectorSubcoreMesh(
        core_axis_name="c", subcore_axis_name="s", num_cores=1
    )
    gather = pl.kernel(
        _gather_body,
        mesh=mesh,
        out_type=jax.ShapeDtypeStruct((_NUM_ROWS, _DIM), jnp.float32),
        scratch_types=[
            pltpu.VMEM((_NUM_CHUNKS, _DIM, 128), jnp.float32),
            pltpu.VMEM((_NUM_ROWS, _DIM), jnp.float32),
            pltpu.SemaphoreType.DMA,
        ],
    )
    return gather(tbl)


# speedup vs baseline: 1.0317x; 1.0008x over previous
import jax
import jax.numpy as jnp
from jax import lax
from jax.experimental import pallas as pl
from jax.experimental.pallas import tpu as pltpu
from jax.experimental.pallas import tpu_sc as plsc

_ROW_INDICES = (5, 8, 7, 16, 256, 123)
_NUM_ROWS = len(_ROW_INDICES)
_DIM = 64
_LANES = 16

_CHUNK_STARTS = tuple(sorted({i - i % 128 for i in _ROW_INDICES}))
_NUM_CHUNKS = len(_CHUNK_STARTS)
_COORDS = tuple(
    (_CHUNK_STARTS.index(i - i % 128), (i % 128) // _LANES, i % _LANES)
    for i in _ROW_INDICES
)
_WINDOWS = tuple(sorted({(blk, win) for blk, win, _ in _COORDS}))


def _gather_body(tbl_hbm, out_hbm, chunks_v, rows_v, sem):
    sid = lax.axis_index("s")

    @pl.when(sid == 0)
    def _():
        copies = [
            pltpu.async_copy(
                tbl_hbm.at[:, pl.ds(start, 128)], chunks_v.at[k], sem
            )
            for k, start in enumerate(_CHUNK_STARTS)
        ]
        for c in copies:
            c.wait()

        pos = lax.iota(jnp.int32, _LANES)

        def q_body(q, _):
            def t_body(t, acc):
                c = q * _LANES + t
                wvals = {
                    (blk, win): chunks_v[blk, c, pl.ds(win * _LANES, _LANES)]
                    for blk, win in _WINDOWS
                }
                return tuple(
                    jnp.where(pos == t, wvals[(blk, win)][lane], a)
                    for a, (blk, win, lane) in zip(acc, _COORDS)
                )

            acc = lax.fori_loop(
                0,
                _LANES,
                t_body,
                tuple(jnp.zeros((_LANES,), jnp.float32) for _ in _ROW_INDICES),
            )
            for j in range(_NUM_ROWS):
                rows_v[j, pl.ds(q * _LANES, _LANES)] = acc[j]
            return 0

        lax.fori_loop(0, _DIM // _LANES, q_body, 0)
        pltpu.sync_copy(rows_v, out_hbm)


def kernel(input):
    tbl = input.T
    mesh = plsc.VectorSubcoreMesh(
        core_axis_name="c", subcore_axis_name="s", num_cores=1
    )
    gather = pl.kernel(
        _gather_body,
        mesh=mesh,
        out_type=jax.ShapeDtypeStruct((_NUM_ROWS, _DIM), jnp.float32),
        scratch_types=[
            pltpu.VMEM((_NUM_CHUNKS, _DIM, 128), jnp.float32),
            pltpu.VMEM((_NUM_ROWS, _DIM), jnp.float32),
            pltpu.SemaphoreType.DMA,
        ],
    )
    return gather(tbl)
